# Initial kernel scaffold; baseline (speedup 1.0000x reference)
import jax
import jax.numpy as jnp
from jax.experimental import pallas as pl


def kernel(x, y, embeddings):
    raise NotImplementedError("write your pallas kernel here")


# speedup vs baseline: 8.9800x; 8.9800x over previous
import functools

import jax
import jax.numpy as jnp
from jax import lax
from jax.experimental import pallas as pl
from jax.experimental.pallas import tpu as pltpu
from jax.experimental.pallas import tpu_sc as plsc

GRID_H, GRID_W, EMBED_DIM = 24, 24, 96
B, P = 1024, 576
N = B * P
NUM_CORES, NUM_SUBCORES = 2, 16
NW = NUM_CORES * NUM_SUBCORES
ROWS_PER_W = N // NW
CHUNK = 128
STEPS = ROWS_PER_W // CHUNK
LANES = 16

_mesh = plsc.VectorSubcoreMesh(core_axis_name="c", subcore_axis_name="s")


@functools.partial(
    pl.kernel,
    mesh=_mesh,
    out_type=jax.ShapeDtypeStruct((N, EMBED_DIM), jnp.float32),
    scratch_types=[
        pltpu.VMEM((ROWS_PER_W,), jnp.int32),
        pltpu.VMEM((ROWS_PER_W,), jnp.int32),
        pltpu.VMEM((2, CHUNK, EMBED_DIM), jnp.float32),
        pltpu.SemaphoreType.DMA,
        pltpu.SemaphoreType.DMA,
        pltpu.SemaphoreType.DMA,
    ],
    compiler_params=pltpu.CompilerParams(use_tc_tiling_on_sc=False),
)
def _sc_gather(x_hbm, y_hbm, table_hbm, out_hbm, x_v, idx_v, rows_v, gsem, ssem0, ssem1):
    wid = lax.axis_index("s") * NUM_CORES + lax.axis_index("c")
    base = wid * ROWS_PER_W

    pltpu.sync_copy(x_hbm.at[pl.ds(base, ROWS_PER_W)], x_v)
    pltpu.sync_copy(y_hbm.at[pl.ds(base, ROWS_PER_W)], idx_v)

    def idx_body(i, _):
        s = pl.ds(i * LANES, LANES)
        idx_v[s] = x_v[s] * GRID_W + idx_v[s]
        return ()

    lax.fori_loop(0, ROWS_PER_W // LANES, idx_body, ())

    ssems = (ssem0, ssem1)

    def gather(j, slot):
        return pltpu.async_copy(
            table_hbm.at[idx_v.at[pl.ds(j * CHUNK, CHUNK)]],
            rows_v.at[slot],
            gsem,
        )

    def store(j, slot):
        pltpu.async_copy(
            rows_v.at[slot],
            out_hbm.at[pl.ds(base + j * CHUNK, CHUNK)],
            ssems[slot],
        )

    def drain_store(slot):
        pltpu.make_async_copy(
            rows_v.at[slot],
            out_hbm.at[pl.ds(base, CHUNK)],
            ssems[slot],
        ).wait()

    gather(0, 0).wait()
    store(0, 0)
    gather(1, 1).wait()
    store(1, 1)

    def body(i, _):
        for slot in (0, 1):
            j = i * 2 + slot
            drain_store(slot)
            gather(j, slot).wait()
            store(j, slot)
        return ()

    lax.fori_loop(1, STEPS // 2, body, ())
    drain_store(0)
    drain_store(1)


def kernel(x, y, embeddings):
    table = embeddings.reshape(GRID_H * GRID_W, EMBED_DIM)
    out = _sc_gather(x.reshape(N), y.reshape(N), table)
    return out.reshape(B, P, EMBED_DIM)
